# baseline (speedup 1.0000x reference)
import jax
import jax.numpy as jnp
from jax import lax
from jax.experimental import pallas as pl
from jax.experimental.pallas import tpu as pltpu
from jax.experimental.pallas import tpu_sc as plsc

N_NODES = 10000
N_EDGES = 320000
DIM = 128
EDIM = 6

NC, NS = 2, 16
NW = NC * NS
EDGES_PER_W = N_EDGES // NW
CHUNK = 80
N_CHUNKS = EDGES_PER_W // CHUNK


def _sc_mesh():
    return plsc.VectorSubcoreMesh(
        core_axis_name="c", subcore_axis_name="s",
        num_cores=NC, num_subcores=NS)


def _support_body(x_ref, w_ref, o_ref):
    o_ref[...] = jnp.dot(x_ref[...], w_ref[...],
                         preferred_element_type=jnp.float32)


def _support(X, W):
    blk = 2000
    return pl.pallas_call(
        _support_body,
        grid=(N_NODES // blk,),
        in_specs=[pl.BlockSpec((blk, DIM), lambda i: (i, 0)),
                  pl.BlockSpec((DIM, DIM), lambda i: (0, 0))],
        out_specs=pl.BlockSpec((blk, DIM), lambda i: (i, 0)),
        out_shape=jax.ShapeDtypeStruct((N_NODES, DIM), jnp.float32),
    )(X, W)


def _gather_body(sup_hbm, src_hbm, out_hbm, idx_v, rows_v, sem):
    cid = lax.axis_index("c")
    sid = lax.axis_index("s")
    base = (cid * NS + sid) * EDGES_PER_W

    def step(j, carry):
        off = base + j * CHUNK
        pltpu.sync_copy(src_hbm.at[pl.ds(off, CHUNK)], idx_v)
        pltpu.async_copy(sup_hbm.at[idx_v], rows_v, sem).wait()
        pltpu.sync_copy(rows_v, out_hbm.at[pl.ds(off, CHUNK)])
        return carry

    lax.fori_loop(0, N_CHUNKS, step, 0)


def _gather(support, src):
    return pl.kernel(
        _gather_body,
        out_type=jax.ShapeDtypeStruct((N_EDGES, DIM), jnp.float32),
        mesh=_sc_mesh(),
        scratch_types=[pltpu.VMEM((CHUNK,), jnp.int32),
                       pltpu.VMEM((CHUNK, DIM), jnp.float32),
                       pltpu.SemaphoreType.DMA],
    )(support, src)


def _edge_body(g_ref, ea_ref, w1e_ref, b1e_ref, w2e_ref, b2e_ref,
               w1sm_ref, w1sd_ref, b1s_ref, w2s_ref, b2s_ref, o_ref):
    ea = ea_ref[...]
    h = jnp.maximum(jnp.dot(ea, w1e_ref[...],
                            preferred_element_type=jnp.float32)
                    + b1e_ref[...], 0.0)
    ef = jnp.dot(h, w2e_ref[...],
                 preferred_element_type=jnp.float32) + b2e_ref[...]
    m = g_ref[...] * ef * (ea[:, 0:1] * ea[:, 1:2])
    hs = (jnp.dot(m, w1sm_ref[...], preferred_element_type=jnp.float32)
          + jnp.dot(ea[:, 2:5], w1sd_ref[...],
                    preferred_element_type=jnp.float32)
          + b1s_ref[...])
    hs = jnp.maximum(hs, 0.0)
    m2 = jnp.dot(hs, w2s_ref[...],
                 preferred_element_type=jnp.float32) + b2s_ref[...]
    o_ref[...] = m2.astype(jnp.bfloat16)


def _edge_chain(g, edge_attr, W1e, b1e, W2e, b2e, W1sm, W1sd, b1s, W2s, b2s):
    blk = 2000
    full = lambda shape: pl.BlockSpec(shape, lambda i: tuple(0 for _ in shape))
    return pl.pallas_call(
        _edge_body,
        grid=(N_EDGES // blk,),
        in_specs=[pl.BlockSpec((blk, DIM), lambda i: (i, 0)),
                  pl.BlockSpec((blk, EDIM), lambda i: (i, 0)),
                  full((EDIM, DIM)), full((1, DIM)),
                  full((DIM, DIM)), full((1, DIM)),
                  full((DIM, DIM)), full((3, DIM)), full((1, DIM)),
                  full((DIM, DIM)), full((1, DIM))],
        out_specs=pl.BlockSpec((blk, DIM), lambda i: (i, 0)),
        out_shape=jax.ShapeDtypeStruct((N_EDGES, DIM), jnp.bfloat16),
    )(g, edge_attr, W1e, b1e, W2e, b2e, W1sm, W1sd, b1s, W2s, b2s)


NB = 2000
EB = 4000


def _tc_scatter_body(tgt_ref, m2_ref, sums_ref, cnt_ref):
    e = pl.program_id(1)

    @pl.when(e == 0)
    def _():
        sums_ref[...] = jnp.zeros_like(sums_ref)
        cnt_ref[...] = jnp.zeros_like(cnt_ref)

    n = pl.program_id(0)
    tgt = tgt_ref[0, 0]
    cols = n * NB + jax.lax.broadcasted_iota(jnp.int32, (EB, NB), 1)
    oh = (tgt[:, None] == cols).astype(jnp.bfloat16)
    dn = (((0,), (0,)), ((), ()))
    sums_ref[...] += jax.lax.dot_general(
        oh, m2_ref[...], dn, preferred_element_type=jnp.float32)
    cnt_ref[...] += jax.lax.dot_general(
        oh, jnp.ones((EB, 8), jnp.bfloat16), dn,
        preferred_element_type=jnp.float32)


def _scatter(m2, tgt):
    tgt2 = tgt.reshape(N_EDGES // EB, 1, EB)
    return pl.pallas_call(
        _tc_scatter_body,
        grid=(N_NODES // NB, N_EDGES // EB),
        in_specs=[pl.BlockSpec((1, 1, EB), lambda n, e: (e, 0, 0)),
                  pl.BlockSpec((EB, DIM), lambda n, e: (e, 0))],
        out_specs=[pl.BlockSpec((NB, DIM), lambda n, e: (n, 0)),
                   pl.BlockSpec((NB, 8), lambda n, e: (n, 0))],
        out_shape=(jax.ShapeDtypeStruct((N_NODES, DIM), jnp.float32),
                   jax.ShapeDtypeStruct((N_NODES, 8), jnp.float32)),
    )(tgt2, m2)


def _final_body(p_ref, d_ref, b_ref, o_ref):
    deg = jnp.maximum(d_ref[:, 0:1], 1.0)
    o_ref[...] = jnp.maximum(p_ref[...] / deg + b_ref[...], 0.0)


def _finalize(parts, degs, bias):
    blk = 2000
    return pl.pallas_call(
        _final_body,
        grid=(N_NODES // blk,),
        in_specs=[pl.BlockSpec((blk, DIM), lambda i: (i, 0)),
                  pl.BlockSpec((blk, 8), lambda i: (i, 0)),
                  pl.BlockSpec((1, DIM), lambda i: (0, 0))],
        out_specs=pl.BlockSpec((blk, DIM), lambda i: (i, 0)),
        out_shape=jax.ShapeDtypeStruct((N_NODES, DIM), jnp.float32),
    )(parts, degs, bias)


def kernel(X, edge_index, edge_attr, W, bias,
           W1e, b1e, W2e, b2e, W1s, b1s, W2s, b2s):
    src = edge_index[0]
    tgt = edge_index[1]
    support = _support(X, W)
    g = _gather(support, src)
    m2 = _edge_chain(g, edge_attr,
                     W1e, b1e.reshape(1, DIM), W2e, b2e.reshape(1, DIM),
                     W1s[:DIM], W1s[DIM:], b1s.reshape(1, DIM),
                     W2s, b2s.reshape(1, DIM))
    sums, cnt = _scatter(m2, tgt)
    return _finalize(sums, cnt, bias.reshape(1, DIM))
